# Initial kernel scaffold; baseline (speedup 1.0000x reference)
import jax
import jax.numpy as jnp
from jax.experimental import pallas as pl


def kernel(x, tables):
    raise NotImplementedError("write your pallas kernel here")


# speedup vs baseline: 1.1388x; 1.1388x over previous
import functools

import jax
import jax.numpy as jnp
from jax import lax
from jax.experimental import pallas as pl
from jax.experimental.pallas import tpu as pltpu
from jax.experimental.pallas import tpu_sc as plsc

_N_LAYERS = 26
_VOCAB = 100000
_DIM = 32
_BATCH = 16384

_NUM_CORES = 2
_NUM_SUBCORES = 16
_NW = _NUM_CORES * _NUM_SUBCORES
_ROWS_PER_GATHER = 128
_TOTAL_ROWS = _BATCH * _N_LAYERS
_PER_W = _TOTAL_ROWS // _NW
_STEPS = _PER_W // _ROWS_PER_GATHER
_IDX_ROWS = _TOTAL_ROWS // _ROWS_PER_GATHER


def _make_sc_gather():
    mesh = plsc.VectorSubcoreMesh(core_axis_name="c", subcore_axis_name="s")

    @functools.partial(
        pl.kernel,
        mesh=mesh,
        out_type=jax.ShapeDtypeStruct((_TOTAL_ROWS, _DIM), jnp.float32),
        scratch_types=[
            pltpu.VMEM((_STEPS, _ROWS_PER_GATHER), jnp.int32),
            pltpu.VMEM((2, _ROWS_PER_GATHER, _DIM), jnp.float32),
            pltpu.SemaphoreType.DMA,
        ],
        compiler_params=pltpu.CompilerParams(use_tc_tiling_on_sc=False),
    )
    def gather_kernel(idx_hbm, table_hbm, out_hbm, idx_v, rows_v, sem):
        wid = lax.axis_index("s") * _NUM_CORES + lax.axis_index("c")
        idx_row0 = wid * _STEPS
        out_row0 = wid * _PER_W
        pltpu.sync_copy(idx_hbm.at[pl.ds(idx_row0, _STEPS)], idx_v)

        def body(j, carry):
            pltpu.async_copy(table_hbm.at[idx_v.at[j]], rows_v.at[0], sem).wait()
            pltpu.sync_copy(
                rows_v.at[0],
                out_hbm.at[pl.ds(out_row0 + j * _ROWS_PER_GATHER, _ROWS_PER_GATHER)],
            )
            return carry

        lax.fori_loop(0, _STEPS, body, 0)

    return gather_kernel


_sc_gather = _make_sc_gather()


def kernel(x, tables):
    offs = (jnp.arange(_N_LAYERS, dtype=jnp.int32) * _VOCAB)[None, :]
    flat_idx = (x + offs).reshape(_IDX_ROWS, _ROWS_PER_GATHER)
    table2d = tables.reshape(_N_LAYERS * _VOCAB, _DIM)
    out = _sc_gather(flat_idx, table2d)
    return out.reshape(_BATCH, _N_LAYERS * _DIM)


# speedup vs baseline: 1.2050x; 1.0581x over previous
import functools

import jax
import jax.numpy as jnp
from jax import lax
from jax.experimental import pallas as pl
from jax.experimental.pallas import tpu as pltpu
from jax.experimental.pallas import tpu_sc as plsc

_N_LAYERS = 26
_VOCAB = 100000
_DIM = 32
_BATCH = 16384

_NUM_CORES = 2
_NUM_SUBCORES = 16
_NW = _NUM_CORES * _NUM_SUBCORES
_ROWS_PER_GATHER = 128
_TOTAL_ROWS = _BATCH * _N_LAYERS
_PER_W = _TOTAL_ROWS // _NW
_STEPS = _PER_W // _ROWS_PER_GATHER
_IDX_ROWS = _TOTAL_ROWS // _ROWS_PER_GATHER


_NBUF = 4
_GROUP = _NBUF
_NK = _STEPS // (2 * _GROUP)


def _make_sc_gather():
    mesh = plsc.VectorSubcoreMesh(core_axis_name="c", subcore_axis_name="s")

    @functools.partial(
        pl.kernel,
        mesh=mesh,
        out_type=jax.ShapeDtypeStruct((_TOTAL_ROWS, _DIM), jnp.float32),
        scratch_types=[
            pltpu.VMEM((_STEPS, _ROWS_PER_GATHER), jnp.int32),
            pltpu.VMEM((2 * _NBUF, _ROWS_PER_GATHER, _DIM), jnp.float32),
            pltpu.SemaphoreType.DMA((2 * _NBUF,)),
            pltpu.SemaphoreType.DMA((2 * _NBUF,)),
        ],
        compiler_params=pltpu.CompilerParams(use_tc_tiling_on_sc=False),
    )
    def gather_kernel(idx_hbm, table_hbm, out_hbm, idx_v, rows_v, gsem, wsem):
        wid = lax.axis_index("s") * _NUM_CORES + lax.axis_index("c")
        idx_row0 = wid * _STEPS
        out_row0 = wid * _PER_W
        pltpu.sync_copy(idx_hbm.at[pl.ds(idx_row0, _STEPS)], idx_v)

        def gather_start(j, b):
            pltpu.async_copy(table_hbm.at[idx_v.at[j]], rows_v.at[b], gsem.at[b])

        def gather_wait(b):
            pltpu.make_async_copy(
                table_hbm.at[pl.ds(0, _ROWS_PER_GATHER)], rows_v.at[b], gsem.at[b]
            ).wait()

        def write_start(j, b):
            pltpu.async_copy(
                rows_v.at[b],
                out_hbm.at[pl.ds(out_row0 + j * _ROWS_PER_GATHER, _ROWS_PER_GATHER)],
                wsem.at[b],
            )

        def write_wait(b):
            pltpu.make_async_copy(
                rows_v.at[b], out_hbm.at[pl.ds(0, _ROWS_PER_GATHER)], wsem.at[b]
            ).wait()

        for i in range(_NBUF):
            gather_start(i, i)

        def body(k, carry):
            odd_base = (2 * k + 1) * _GROUP

            @pl.when(k > 0)
            def _():
                for i in range(_NBUF):
                    write_wait(_NBUF + i)

            for i in range(_NBUF):
                gather_start(odd_base + i, _NBUF + i)

            for i in range(_NBUF):
                gather_wait(i)
                write_start(2 * k * _GROUP + i, i)

            @pl.when(k < _NK - 1)
            def _():
                for i in range(_NBUF):
                    write_wait(i)
                    gather_start((2 * k + 2) * _GROUP + i, i)

            for i in range(_NBUF):
                gather_wait(_NBUF + i)
                write_start(odd_base + i, _NBUF + i)
            return carry

        lax.fori_loop(0, _NK, body, 0)

        for i in range(2 * _NBUF):
            write_wait(i)

    return gather_kernel


_sc_gather = _make_sc_gather()


def kernel(x, tables):
    offs = (jnp.arange(_N_LAYERS, dtype=jnp.int32) * _VOCAB)[None, :]
    flat_idx = (x + offs).reshape(_IDX_ROWS, _ROWS_PER_GATHER)
    table2d = tables.reshape(_N_LAYERS * _VOCAB, _DIM)
    out = _sc_gather(flat_idx, table2d)
    return out.reshape(_BATCH, _N_LAYERS * _DIM)
